# baseline (speedup 1.0000x reference)
import functools

import jax
import jax.numpy as jnp
from jax import lax
from jax.experimental import pallas as pl
from jax.experimental.pallas import tpu as pltpu
from jax.experimental.pallas import tpu_sc as plsc

E = 8
TOPK = 2
H = 1024
F = 4096
T = 4096
P = T * TOPK
CAP = 1024

NC, NS, L = 2, 16, 16
NW = NC * NS
PPW = P // NW
BLK = 2 * PPW
CH = 32
NCH = PPW // CH
TPW = T // NW
TPC = CH // TOPK


@functools.cache
def _sc_mesh():
    return plsc.VectorSubcoreMesh(core_axis_name="c", subcore_axis_name="s",
                                  num_cores=NC, num_subcores=NS)


def _prefix16(x, sb):
    acc = x
    for k in (1, 2, 4, 8):
        sb[pl.ds(L, L)] = acc
        acc = acc + sb[pl.ds(L - k, L)]
    return acc


def _route_dispatch_body(te_hbm, ew_hbm, x_hbm, xs_hbm, d_hbm, w_hbm,
                         te_v, ranks_v, ew_v, row_v, tab_v, sb_v, hist_sh,
                         dd_v, tok_v, dout_v, wout_v, rows_v, rows2_v, zrow_v,
                         gsem, ssem):
    cid = lax.axis_index("c")
    sid = lax.axis_index("s")
    g = 2 * sid + cid
    lanes = lax.iota(jnp.int32, L)
    zeros = jnp.zeros((L,), jnp.int32)

    blk_base = sid * BLK
    pltpu.sync_copy(te_hbm.at[pl.ds(blk_base, BLK)], te_v)
    sb_v[pl.ds(0, L)] = zeros
    cnt = [jnp.int32(0)] * E
    for j in range(BLK // L):
        v = te_v[pl.ds(j * L, L)]
        lo = v < 4
        sh1 = jnp.minimum(v, 3) * 8
        sh2 = jnp.maximum(v - 4, 0) * 8
        pv1 = jnp.where(lo, jnp.int32(1) << sh1, zeros)
        pv2 = jnp.where(lo, zeros, jnp.int32(1) << sh2)
        pre1 = _prefix16(pv1, sb_v)
        pre2 = _prefix16(pv2, sb_v)
        fld = jnp.where(lo, pre1 >> sh1, pre2 >> sh2) & 0xFF
        carry = jnp.full((L,), cnt[7], jnp.int32)
        for e in range(E - 1):
            carry = jnp.where(v == e, jnp.full((L,), cnt[e], jnp.int32), carry)
        ranks_v[pl.ds(j * L, L)] = fld - 1 + carry
        l1, l2 = pre1[L - 1], pre2[L - 1]
        for e in range(4):
            cnt[e] = cnt[e] + ((l1 >> (8 * e)) & 0xFF)
            cnt[e + 4] = cnt[e + 4] + ((l2 >> (8 * e)) & 0xFF)
    hist = jnp.full((L,), 0, jnp.int32)
    for e in range(E):
        hist = jnp.where(lanes == e, cnt[e], hist)
    row_v[...] = hist
    pltpu.sync_copy(row_v, hist_sh.at[sid])
    plsc.subcore_barrier()
    pltpu.sync_copy(hist_sh, tab_v)

    totals = zeros
    pfx = zeros
    sidv = jnp.full((L,), sid, jnp.int32)
    ones = jnp.full((L,), 1, jnp.int32)
    for s in range(NS):
        r = tab_v[s]
        totals = totals + r
        step = jnp.minimum(jnp.maximum(sidv - jnp.full((L,), s, jnp.int32),
                                       zeros), ones)
        pfx = pfx + r * step
    off = [None] * E
    cap_off = [None] * E
    tot0 = totals[0]
    for e in range(E):
        p_e = pfx[e]
        off[e] = e * CAP + p_e
        cap_off[e] = p_e

    loc = cid * PPW
    pltpu.sync_copy(ew_hbm.at[pl.ds(blk_base + loc, PPW)], ew_v)
    for j in range(PPW // L):
        v = te_v[pl.ds(loc + j * L, L)]
        r = ranks_v[pl.ds(loc + j * L, L)]
        ewv = ew_v[pl.ds(j * L, L)]
        offs = jnp.full((L,), off[7], jnp.int32)
        pfxs = jnp.full((L,), cap_off[7], jnp.int32)
        for e in range(E - 1):
            m = v == e
            offs = jnp.where(m, jnp.full((L,), off[e], jnp.int32), offs)
            pfxs = jnp.where(m, jnp.full((L,), cap_off[e], jnp.int32), pfxs)
        d = offs + r
        kept = (pfxs + r) < CAP
        dd = jnp.where(kept, d, jnp.full((L,), P + g, jnp.int32))
        dout = jnp.where(kept, d, jnp.zeros((L,), jnp.int32))
        wout = jnp.where(kept, ewv, jnp.zeros((L,), jnp.float32))
        tok = lax.shift_right_logical(blk_base + loc + j * L + lanes, 1)
        k, half = j // 2, (j % 2) * L
        dd_v[k, pl.ds(half, L)] = dd
        tok_v[k, pl.ds(half, L)] = tok
        dout_v[k, pl.ds(half, L)] = dout
        wout_v[pl.ds(j * L, L)] = wout
    pltpu.sync_copy(dout_v, d_hbm.at[g])
    pltpu.sync_copy(wout_v, w_hbm.at[g])

    for i in range(H // L):
        zrow_v[0, pl.ds(i * L, L)] = jnp.zeros((L,), jnp.float32)

    @pl.when(jnp.logical_and(g == 0, tot0 == 0))
    def _():
        pltpu.sync_copy(zrow_v, xs_hbm.at[pl.ds(0, 1)])

    bufs = (rows_v, rows2_v)
    gd = [None] * NCH
    sd = [None] * NCH
    gd[0] = pltpu.async_copy(x_hbm.at[tok_v.at[0]], bufs[0], gsem)
    for k in range(NCH):
        gd[k].wait()
        sd[k] = pltpu.async_copy(bufs[k % 2].at[:, 0],
                                 xs_hbm.at[dd_v.at[k]], ssem)
        if k >= 1:
            sd[k - 1].wait()
        if k + 1 < NCH:
            gd[k + 1] = pltpu.async_copy(x_hbm.at[tok_v.at[k + 1]],
                                         bufs[(k + 1) % 2], gsem)
    sd[NCH - 1].wait()


def _route_dispatch(te, ew, xf):
    return pl.kernel(
        _route_dispatch_body,
        out_type=[
            jax.ShapeDtypeStruct((P + NW, H), jnp.float32),
            jax.ShapeDtypeStruct((NW, NCH, CH), jnp.int32),
            jax.ShapeDtypeStruct((NW, PPW), jnp.float32),
        ],
        mesh=_sc_mesh(),
        scratch_types=[
            pltpu.VMEM((BLK,), jnp.int32),
            pltpu.VMEM((BLK,), jnp.int32),
            pltpu.VMEM((PPW,), jnp.float32),
            pltpu.VMEM((L,), jnp.int32),
            pltpu.VMEM((NS, L), jnp.int32),
            pltpu.VMEM((2 * L,), jnp.int32),
            pltpu.VMEM_SHARED((NS, L), jnp.int32),
            pltpu.VMEM((NCH, CH), jnp.int32),
            pltpu.VMEM((NCH, CH), jnp.int32),
            pltpu.VMEM((NCH, CH), jnp.int32),
            pltpu.VMEM((PPW,), jnp.float32),
            pltpu.VMEM((CH, 1, H), jnp.float32),
            pltpu.VMEM((CH, 1, H), jnp.float32),
            pltpu.VMEM((1, H), jnp.float32),
            pltpu.SemaphoreType.DMA,
            pltpu.SemaphoreType.DMA,
        ],
    )(te, ew, xf)


FB = 512
FN = F // FB


def _mlp_body(xs_ref, w1_ref, w2_ref, y_ref, acc_ref):
    f = pl.program_id(1)
    a = xs_ref[...].astype(jnp.bfloat16)
    h = jnp.dot(a, w1_ref[0].astype(jnp.bfloat16),
                preferred_element_type=jnp.float32)
    hb = jax.nn.gelu(h).astype(jnp.bfloat16)
    part = jnp.dot(hb, w2_ref[0].astype(jnp.bfloat16),
                   preferred_element_type=jnp.float32)

    @pl.when(f == 0)
    def _():
        acc_ref[...] = jnp.zeros_like(acc_ref)

    acc_ref[...] += part

    @pl.when(f == FN - 1)
    def _():
        y_ref[...] = acc_ref[...]


def _grouped_mlp(xs, w1, w2):
    return pl.pallas_call(
        _mlp_body,
        grid=(E, FN),
        in_specs=[
            pl.BlockSpec((CAP, H), lambda e, f: (e, 0)),
            pl.BlockSpec((1, H, FB), lambda e, f: (e, 0, f)),
            pl.BlockSpec((1, FB, H), lambda e, f: (e, f, 0)),
        ],
        out_specs=pl.BlockSpec((CAP, H), lambda e, f: (e, 0)),
        out_shape=jax.ShapeDtypeStruct((P, H), jnp.float32),
        scratch_shapes=[pltpu.VMEM((CAP, H), jnp.float32)],
        compiler_params=pltpu.CompilerParams(
            dimension_semantics=("parallel", "arbitrary"),
        ),
    )(xs, w1, w2)


VPR = H // L


def _sc_combine_body(y_hbm, d_hbm, w_hbm, out_hbm, d_v, w_v, rows_v, out_v,
                     sem):
    cid = lax.axis_index("c")
    sid = lax.axis_index("s")
    g = 2 * sid + cid
    pltpu.sync_copy(d_hbm.at[g], d_v)
    pltpu.sync_copy(w_hbm.at[g], w_v.at[pl.ds(0, PPW)])

    pltpu.async_copy(y_hbm.at[d_v.at[0]], rows_v.at[pl.ds(0, CH)], sem)

    def kstep(k, carry):
        half = (k & 1) * CH
        pltpu.make_async_copy(y_hbm.at[d_v.at[k]],
                              rows_v.at[pl.ds(half, CH)], sem).wait()

        @pl.when(k + 1 < NCH)
        def _():
            nhalf = ((k + 1) & 1) * CH
            pltpu.async_copy(y_hbm.at[d_v.at[k + 1]],
                             rows_v.at[pl.ds(nhalf, CH)], sem)

        for t in range(TPC):
            wv = w_v[pl.ds(k * CH + 2 * t, L)]
            w0 = wv[0]
            w1 = wv[1]
            for i in range(VPR):
                r0 = rows_v[half + 2 * t, pl.ds(i * L, L)]
                r1 = rows_v[half + 2 * t + 1, pl.ds(i * L, L)]
                out_v[t, 0, pl.ds(i * L, L)] = w0 * r0 + w1 * r1
        pltpu.sync_copy(out_v, out_hbm.at[pl.ds(g * TPW + k * TPC, TPC)])
        return carry

    lax.fori_loop(0, NCH, kstep, jnp.int32(0))


def _sc_combine(y, d, w):
    return pl.kernel(
        _sc_combine_body,
        out_type=jax.ShapeDtypeStruct((T, 1, H), jnp.float32),
        mesh=_sc_mesh(),
        scratch_types=[
            pltpu.VMEM((NCH, CH), jnp.int32),
            pltpu.VMEM((PPW + L,), jnp.float32),
            pltpu.VMEM((2 * CH, H), jnp.float32),
            pltpu.VMEM((TPC, 1, H), jnp.float32),
            pltpu.SemaphoreType.DMA,
        ],
    )(y, d, w)


def kernel(x, scores, expert_weights, top_experts, w1, w2):
    del scores
    te = top_experts.reshape(-1).astype(jnp.int32)
    ew = expert_weights.reshape(-1)
    xs, d, w = _route_dispatch(te, ew, x)
    y = _grouped_mlp(xs, w1, w2)
    return _sc_combine(y, d, w)


# speedup vs baseline: 1.1204x; 1.1204x over previous
import functools

import jax
import jax.numpy as jnp
from jax import lax
from jax.experimental import pallas as pl
from jax.experimental.pallas import tpu as pltpu
from jax.experimental.pallas import tpu_sc as plsc

E = 8
TOPK = 2
H = 1024
F = 4096
T = 4096
P = T * TOPK
CAP = 1024

NC, NS, L = 2, 16, 16
NW = NC * NS
PPW = P // NW
BLK = 2 * PPW
CH = 32
NCH = PPW // CH
TPW = T // NW
TPC = CH // TOPK


@functools.cache
def _sc_mesh():
    return plsc.VectorSubcoreMesh(core_axis_name="c", subcore_axis_name="s",
                                  num_cores=NC, num_subcores=NS)


def _prefix16(x, sb):
    acc = x
    for k in (1, 2, 4, 8):
        sb[pl.ds(L, L)] = acc
        acc = acc + sb[pl.ds(L - k, L)]
    return acc


def _route_dispatch_body(te_hbm, ew_hbm, x_hbm, xs_hbm, d_hbm, w_hbm,
                         te_v, ranks_v, ew_v, row_v, tab_v, sb_v, hist_sh,
                         dd_v, tok_v, dout_v, wout_v, rows_v, rows2_v, zrow_v,
                         gsem, ssem):
    cid = lax.axis_index("c")
    sid = lax.axis_index("s")
    g = 2 * sid + cid
    lanes = lax.iota(jnp.int32, L)
    zeros = jnp.zeros((L,), jnp.int32)

    blk_base = sid * BLK
    pltpu.sync_copy(te_hbm.at[pl.ds(blk_base, BLK)], te_v)
    sb_v[pl.ds(0, L)] = zeros
    cnt = [jnp.int32(0)] * E
    for j in range(BLK // L):
        v = te_v[pl.ds(j * L, L)]
        lo = v < 4
        sh1 = jnp.minimum(v, 3) * 8
        sh2 = jnp.maximum(v - 4, 0) * 8
        pv1 = jnp.where(lo, jnp.int32(1) << sh1, zeros)
        pv2 = jnp.where(lo, zeros, jnp.int32(1) << sh2)
        pre1 = _prefix16(pv1, sb_v)
        pre2 = _prefix16(pv2, sb_v)
        fld = jnp.where(lo, pre1 >> sh1, pre2 >> sh2) & 0xFF
        carry = jnp.full((L,), cnt[7], jnp.int32)
        for e in range(E - 1):
            carry = jnp.where(v == e, jnp.full((L,), cnt[e], jnp.int32), carry)
        ranks_v[pl.ds(j * L, L)] = fld - 1 + carry
        l1, l2 = pre1[L - 1], pre2[L - 1]
        for e in range(4):
            cnt[e] = cnt[e] + ((l1 >> (8 * e)) & 0xFF)
            cnt[e + 4] = cnt[e + 4] + ((l2 >> (8 * e)) & 0xFF)
    hist = jnp.full((L,), 0, jnp.int32)
    for e in range(E):
        hist = jnp.where(lanes == e, cnt[e], hist)
    row_v[...] = hist
    pltpu.sync_copy(row_v, hist_sh.at[sid])
    plsc.subcore_barrier()
    pltpu.sync_copy(hist_sh, tab_v)

    totals = zeros
    pfx = zeros
    sidv = jnp.full((L,), sid, jnp.int32)
    ones = jnp.full((L,), 1, jnp.int32)
    for s in range(NS):
        r = tab_v[s]
        totals = totals + r
        step = jnp.minimum(jnp.maximum(sidv - jnp.full((L,), s, jnp.int32),
                                       zeros), ones)
        pfx = pfx + r * step
    off = [None] * E
    cap_off = [None] * E
    tot0 = totals[0]
    for e in range(E):
        p_e = pfx[e]
        off[e] = e * CAP + p_e
        cap_off[e] = p_e

    loc = cid * PPW
    pltpu.sync_copy(ew_hbm.at[pl.ds(blk_base + loc, PPW)], ew_v)
    for j in range(PPW // L):
        v = te_v[pl.ds(loc + j * L, L)]
        r = ranks_v[pl.ds(loc + j * L, L)]
        ewv = ew_v[pl.ds(j * L, L)]
        offs = jnp.full((L,), off[7], jnp.int32)
        pfxs = jnp.full((L,), cap_off[7], jnp.int32)
        for e in range(E - 1):
            m = v == e
            offs = jnp.where(m, jnp.full((L,), off[e], jnp.int32), offs)
            pfxs = jnp.where(m, jnp.full((L,), cap_off[e], jnp.int32), pfxs)
        d = offs + r
        kept = (pfxs + r) < CAP
        dd = jnp.where(kept, d, jnp.full((L,), P + g, jnp.int32))
        dout = jnp.where(kept, d, jnp.zeros((L,), jnp.int32))
        wout = jnp.where(kept, ewv, jnp.zeros((L,), jnp.float32))
        tok = lax.shift_right_logical(blk_base + loc + j * L + lanes, 1)
        k, half = j // 2, (j % 2) * L
        dd_v[k, pl.ds(half, L)] = dd
        tok_v[k, pl.ds(half, L)] = tok
        dout_v[k, pl.ds(half, L)] = dout
        wout_v[pl.ds(j * L, L)] = wout
    pltpu.sync_copy(dout_v, d_hbm.at[g])
    pltpu.sync_copy(wout_v, w_hbm.at[g])

    for i in range(H // L):
        zrow_v[0, pl.ds(i * L, L)] = jnp.zeros((L,), jnp.float32)

    @pl.when(jnp.logical_and(g == 0, tot0 == 0))
    def _():
        pltpu.sync_copy(zrow_v, xs_hbm.at[pl.ds(0, 1)])

    bufs = (rows_v, rows2_v)
    gd = [None] * NCH
    sd = [None] * NCH
    gd[0] = pltpu.async_copy(x_hbm.at[tok_v.at[0]], bufs[0], gsem)
    for k in range(NCH):
        gd[k].wait()
        sd[k] = pltpu.async_copy(bufs[k % 2].at[:, 0],
                                 xs_hbm.at[dd_v.at[k]], ssem)
        if k >= 1:
            sd[k - 1].wait()
        if k + 1 < NCH:
            gd[k + 1] = pltpu.async_copy(x_hbm.at[tok_v.at[k + 1]],
                                         bufs[(k + 1) % 2], gsem)
    sd[NCH - 1].wait()


def _route_dispatch(te, ew, xf):
    return pl.kernel(
        _route_dispatch_body,
        out_type=[
            jax.ShapeDtypeStruct((P + NW, H), jnp.float32),
            jax.ShapeDtypeStruct((NW, NCH, CH), jnp.int32),
            jax.ShapeDtypeStruct((NW, PPW), jnp.float32),
        ],
        mesh=_sc_mesh(),
        scratch_types=[
            pltpu.VMEM((BLK,), jnp.int32),
            pltpu.VMEM((BLK,), jnp.int32),
            pltpu.VMEM((PPW,), jnp.float32),
            pltpu.VMEM((L,), jnp.int32),
            pltpu.VMEM((NS, L), jnp.int32),
            pltpu.VMEM((2 * L,), jnp.int32),
            pltpu.VMEM_SHARED((NS, L), jnp.int32),
            pltpu.VMEM((NCH, CH), jnp.int32),
            pltpu.VMEM((NCH, CH), jnp.int32),
            pltpu.VMEM((NCH, CH), jnp.int32),
            pltpu.VMEM((PPW,), jnp.float32),
            pltpu.VMEM((CH, 1, H), jnp.float32),
            pltpu.VMEM((CH, 1, H), jnp.float32),
            pltpu.VMEM((1, H), jnp.float32),
            pltpu.SemaphoreType.DMA,
            pltpu.SemaphoreType.DMA,
        ],
    )(te, ew, xf)


FB = 512
FN = F // FB


def _mlp_body(xs_ref, w1_ref, w2_ref, y_ref, acc_ref):
    f = pl.program_id(1)
    a = xs_ref[...].astype(jnp.bfloat16)
    h = jnp.dot(a, w1_ref[0].astype(jnp.bfloat16),
                preferred_element_type=jnp.float32)
    hb = jax.nn.gelu(h).astype(jnp.bfloat16)
    part = jnp.dot(hb, w2_ref[0].astype(jnp.bfloat16),
                   preferred_element_type=jnp.float32)

    @pl.when(f == 0)
    def _():
        acc_ref[...] = jnp.zeros_like(acc_ref)

    acc_ref[...] += part

    @pl.when(f == FN - 1)
    def _():
        y_ref[...] = acc_ref[...]


def _grouped_mlp(xs, w1, w2):
    return pl.pallas_call(
        _mlp_body,
        grid=(E, FN),
        in_specs=[
            pl.BlockSpec((CAP, H), lambda e, f: (e, 0)),
            pl.BlockSpec((1, H, FB), lambda e, f: (e, 0, f)),
            pl.BlockSpec((1, FB, H), lambda e, f: (e, f, 0)),
        ],
        out_specs=pl.BlockSpec((CAP, H), lambda e, f: (e, 0)),
        out_shape=jax.ShapeDtypeStruct((P, H), jnp.float32),
        scratch_shapes=[pltpu.VMEM((CAP, H), jnp.float32)],
        compiler_params=pltpu.CompilerParams(
            dimension_semantics=("parallel", "arbitrary"),
        ),
    )(xs, w1, w2)


VPR = H // L


def _sc_combine_body(y_hbm, d_hbm, w_hbm, out_hbm, d_v, w_v, rows_v, out_v,
                     sem):
    cid = lax.axis_index("c")
    sid = lax.axis_index("s")
    g = 2 * sid + cid
    pltpu.sync_copy(d_hbm.at[g], d_v)
    pltpu.sync_copy(w_hbm.at[g], w_v.at[pl.ds(0, PPW)])

    def kstep(k, carry):
        pltpu.async_copy(y_hbm.at[d_v.at[k]], rows_v, sem).wait()
        for t in range(TPC):
            wv = w_v[pl.ds(k * CH + 2 * t, L)]
            w0 = wv[0]
            w1 = wv[1]
            for i in range(VPR):
                r0 = rows_v[2 * t, pl.ds(i * L, L)]
                r1 = rows_v[2 * t + 1, pl.ds(i * L, L)]
                out_v[t, 0, pl.ds(i * L, L)] = w0 * r0 + w1 * r1
        pltpu.sync_copy(out_v, out_hbm.at[pl.ds(g * TPW + k * TPC, TPC)])
        return carry

    lax.fori_loop(0, NCH, kstep, jnp.int32(0))


def _sc_combine(y, d, w):
    return pl.kernel(
        _sc_combine_body,
        out_type=jax.ShapeDtypeStruct((T, 1, H), jnp.float32),
        mesh=_sc_mesh(),
        scratch_types=[
            pltpu.VMEM((NCH, CH), jnp.int32),
            pltpu.VMEM((PPW + L,), jnp.float32),
            pltpu.VMEM((CH, H), jnp.float32),
            pltpu.VMEM((TPC, 1, H), jnp.float32),
            pltpu.SemaphoreType.DMA,
        ],
    )(y, d, w)


def kernel(x, scores, expert_weights, top_experts, w1, w2):
    del scores
    te = top_experts.reshape(-1).astype(jnp.int32)
    ew = expert_weights.reshape(-1)
    xs, d, w = _route_dispatch(te, ew, x)
    y = _grouped_mlp(xs, w1, w2)
    return _sc_combine(y, d, w)


# speedup vs baseline: 1.1600x; 1.0353x over previous
import functools

import jax
import jax.numpy as jnp
from jax import lax
from jax.experimental import pallas as pl
from jax.experimental.pallas import tpu as pltpu
from jax.experimental.pallas import tpu_sc as plsc

E = 8
TOPK = 2
H = 1024
F = 4096
T = 4096
P = T * TOPK
CAP = 1024

NC, NS, L = 2, 16, 16
NW = NC * NS
PPW = P // NW
BLK = 2 * PPW
CH = 32
NCH = PPW // CH
TPW = T // NW
TPC = CH // TOPK


@functools.cache
def _sc_mesh():
    return plsc.VectorSubcoreMesh(core_axis_name="c", subcore_axis_name="s",
                                  num_cores=NC, num_subcores=NS)


def _prefix16(x, sb):
    acc = x
    for k in (1, 2, 4, 8):
        sb[pl.ds(L, L)] = acc
        acc = acc + sb[pl.ds(L - k, L)]
    return acc


def _route_dispatch_body(te_hbm, ew_hbm, x_hbm, xs_hbm, d_hbm, w_hbm,
                         te_v, ranks_v, ew_v, row_v, tab_v, sb_v, hist_sh,
                         dd_v, tok_v, dout_v, wout_v, rows_v, rows2_v, zrow_v,
                         gsem, ssem):
    cid = lax.axis_index("c")
    sid = lax.axis_index("s")
    g = 2 * sid + cid
    lanes = lax.iota(jnp.int32, L)
    zeros = jnp.zeros((L,), jnp.int32)

    blk_base = sid * BLK
    pltpu.sync_copy(te_hbm.at[pl.ds(blk_base, BLK)], te_v)
    sb_v[pl.ds(0, L)] = zeros
    cnt = [jnp.int32(0)] * E
    for j in range(BLK // L):
        v = te_v[pl.ds(j * L, L)]
        lo = v < 4
        sh1 = jnp.minimum(v, 3) * 8
        sh2 = jnp.maximum(v - 4, 0) * 8
        pv1 = jnp.where(lo, jnp.int32(1) << sh1, zeros)
        pv2 = jnp.where(lo, zeros, jnp.int32(1) << sh2)
        pre1 = _prefix16(pv1, sb_v)
        pre2 = _prefix16(pv2, sb_v)
        fld = jnp.where(lo, pre1 >> sh1, pre2 >> sh2) & 0xFF
        carry = jnp.full((L,), cnt[7], jnp.int32)
        for e in range(E - 1):
            carry = jnp.where(v == e, jnp.full((L,), cnt[e], jnp.int32), carry)
        ranks_v[pl.ds(j * L, L)] = fld - 1 + carry
        l1, l2 = pre1[L - 1], pre2[L - 1]
        for e in range(4):
            cnt[e] = cnt[e] + ((l1 >> (8 * e)) & 0xFF)
            cnt[e + 4] = cnt[e + 4] + ((l2 >> (8 * e)) & 0xFF)
    hist = jnp.full((L,), 0, jnp.int32)
    for e in range(E):
        hist = jnp.where(lanes == e, cnt[e], hist)
    row_v[...] = hist
    pltpu.sync_copy(row_v, hist_sh.at[sid])
    plsc.subcore_barrier()
    pltpu.sync_copy(hist_sh, tab_v)

    totals = zeros
    pfx = zeros
    sidv = jnp.full((L,), sid, jnp.int32)
    ones = jnp.full((L,), 1, jnp.int32)
    for s in range(NS):
        r = tab_v[s]
        totals = totals + r
        step = jnp.minimum(jnp.maximum(sidv - jnp.full((L,), s, jnp.int32),
                                       zeros), ones)
        pfx = pfx + r * step
    off = [None] * E
    cap_off = [None] * E
    tot0 = totals[0]
    for e in range(E):
        p_e = pfx[e]
        off[e] = e * CAP + p_e
        cap_off[e] = p_e

    loc = cid * PPW
    pltpu.sync_copy(ew_hbm.at[pl.ds(blk_base + loc, PPW)], ew_v)
    for j in range(PPW // L):
        v = te_v[pl.ds(loc + j * L, L)]
        r = ranks_v[pl.ds(loc + j * L, L)]
        ewv = ew_v[pl.ds(j * L, L)]
        offs = jnp.full((L,), off[7], jnp.int32)
        pfxs = jnp.full((L,), cap_off[7], jnp.int32)
        for e in range(E - 1):
            m = v == e
            offs = jnp.where(m, jnp.full((L,), off[e], jnp.int32), offs)
            pfxs = jnp.where(m, jnp.full((L,), cap_off[e], jnp.int32), pfxs)
        d = offs + r
        kept = (pfxs + r) < CAP
        dd = jnp.where(kept, d, jnp.full((L,), P + g, jnp.int32))
        dout = jnp.where(kept, d, jnp.zeros((L,), jnp.int32))
        wout = jnp.where(kept, ewv, jnp.zeros((L,), jnp.float32))
        tok = lax.shift_right_logical(blk_base + loc + j * L + lanes, 1)
        k, half = j // 2, (j % 2) * L
        dd_v[k, pl.ds(half, L)] = dd
        tok_v[k, pl.ds(half, L)] = tok
        dout_v[k, pl.ds(half, L)] = dout
        wout_v[pl.ds(j * L, L)] = wout
    pltpu.sync_copy(dout_v, d_hbm.at[g])
    pltpu.sync_copy(wout_v, w_hbm.at[g])

    for i in range(H // L):
        zrow_v[0, pl.ds(i * L, L)] = jnp.zeros((L,), jnp.float32)

    @pl.when(jnp.logical_and(g == 0, tot0 == 0))
    def _():
        pltpu.sync_copy(zrow_v, xs_hbm.at[pl.ds(0, 1)])

    bufs = (rows_v, rows2_v)
    gd = [None] * NCH
    sd = [None] * NCH
    gd[0] = pltpu.async_copy(x_hbm.at[tok_v.at[0]], bufs[0], gsem)
    for k in range(NCH):
        gd[k].wait()
        sd[k] = pltpu.async_copy(bufs[k % 2].at[:, 0],
                                 xs_hbm.at[dd_v.at[k]], ssem)
        if k >= 1:
            sd[k - 1].wait()
        if k + 1 < NCH:
            gd[k + 1] = pltpu.async_copy(x_hbm.at[tok_v.at[k + 1]],
                                         bufs[(k + 1) % 2], gsem)
    sd[NCH - 1].wait()


def _route_dispatch(te, ew, xf):
    return pl.kernel(
        _route_dispatch_body,
        out_type=[
            jax.ShapeDtypeStruct((P + NW, H), jnp.float32),
            jax.ShapeDtypeStruct((NW, NCH, CH), jnp.int32),
            jax.ShapeDtypeStruct((NW, PPW), jnp.float32),
        ],
        mesh=_sc_mesh(),
        scratch_types=[
            pltpu.VMEM((BLK,), jnp.int32),
            pltpu.VMEM((BLK,), jnp.int32),
            pltpu.VMEM((PPW,), jnp.float32),
            pltpu.VMEM((L,), jnp.int32),
            pltpu.VMEM((NS, L), jnp.int32),
            pltpu.VMEM((2 * L,), jnp.int32),
            pltpu.VMEM_SHARED((NS, L), jnp.int32),
            pltpu.VMEM((NCH, CH), jnp.int32),
            pltpu.VMEM((NCH, CH), jnp.int32),
            pltpu.VMEM((NCH, CH), jnp.int32),
            pltpu.VMEM((PPW,), jnp.float32),
            pltpu.VMEM((CH, 1, H), jnp.float32),
            pltpu.VMEM((CH, 1, H), jnp.float32),
            pltpu.VMEM((1, H), jnp.float32),
            pltpu.SemaphoreType.DMA,
            pltpu.SemaphoreType.DMA,
        ],
    )(te, ew, xf)


FB = 512
FN = F // FB


def _mlp_body(xs_ref, w1_ref, w2_ref, y_ref, acc_ref):
    f = pl.program_id(1)
    a = xs_ref[...].astype(jnp.bfloat16)
    h = jnp.dot(a, w1_ref[0].astype(jnp.bfloat16),
                preferred_element_type=jnp.float32)
    hb = jax.nn.gelu(h).astype(jnp.bfloat16)
    part = jnp.dot(hb, w2_ref[0].astype(jnp.bfloat16),
                   preferred_element_type=jnp.float32)

    @pl.when(f == 0)
    def _():
        acc_ref[...] = jnp.zeros_like(acc_ref)

    acc_ref[...] += part

    @pl.when(f == FN - 1)
    def _():
        y_ref[...] = acc_ref[...]


def _grouped_mlp(xs, w1, w2):
    return pl.pallas_call(
        _mlp_body,
        grid=(E, FN),
        in_specs=[
            pl.BlockSpec((CAP, H), lambda e, f: (e, 0)),
            pl.BlockSpec((1, H, FB), lambda e, f: (e, 0, f)),
            pl.BlockSpec((1, FB, H), lambda e, f: (e, f, 0)),
        ],
        out_specs=pl.BlockSpec((CAP, H), lambda e, f: (e, 0)),
        out_shape=jax.ShapeDtypeStruct((P, H), jnp.float32),
        scratch_shapes=[pltpu.VMEM((CAP, H), jnp.float32)],
        compiler_params=pltpu.CompilerParams(
            dimension_semantics=("parallel", "arbitrary"),
        ),
    )(xs, w1, w2)


VPR = H // L


CC = 16
NCC = PPW // CC
TPQ = CC // TOPK


def _sc_combine_body(y_hbm, d_hbm, w_hbm, out_hbm, d_v, w_v, bufa_v, bufb_v,
                     out_v, sema, semb):
    cid = lax.axis_index("c")
    sid = lax.axis_index("s")
    g = 2 * sid + cid
    pltpu.sync_copy(d_hbm.at[g], d_v)
    pltpu.sync_copy(w_hbm.at[g], w_v.at[pl.ds(0, PPW)])

    def compute(buf, k):
        for t in range(TPQ):
            wv = w_v[pl.ds(k * CC + 2 * t, L)]
            w0 = wv[0]
            w1 = wv[1]
            for i in range(VPR):
                r0 = buf[2 * t, pl.ds(i * L, L)]
                r1 = buf[2 * t + 1, pl.ds(i * L, L)]
                out_v[t, 0, pl.ds(i * L, L)] = w0 * r0 + w1 * r1
        pltpu.sync_copy(out_v, out_hbm.at[pl.ds(g * TPW + k * TPQ, TPQ)])

    pltpu.async_copy(y_hbm.at[d_v.at[0]], bufa_v, sema)

    def mstep(m, carry):
        k0 = 2 * m
        k1 = 2 * m + 1
        pltpu.async_copy(y_hbm.at[d_v.at[k1]], bufb_v, semb)
        pltpu.make_async_copy(y_hbm.at[d_v.at[k0]], bufa_v, sema).wait()
        compute(bufa_v, k0)

        @pl.when(k0 + 2 < NCC)
        def _():
            pltpu.async_copy(y_hbm.at[d_v.at[k0 + 2]], bufa_v, sema)

        pltpu.make_async_copy(y_hbm.at[d_v.at[k1]], bufb_v, semb).wait()
        compute(bufb_v, k1)
        return carry

    lax.fori_loop(0, NCC // 2, mstep, jnp.int32(0))


def _sc_combine(y, d, w):
    return pl.kernel(
        _sc_combine_body,
        out_type=jax.ShapeDtypeStruct((T, 1, H), jnp.float32),
        mesh=_sc_mesh(),
        scratch_types=[
            pltpu.VMEM((NCC, CC), jnp.int32),
            pltpu.VMEM((PPW + L,), jnp.float32),
            pltpu.VMEM((CC, H), jnp.float32),
            pltpu.VMEM((CC, H), jnp.float32),
            pltpu.VMEM((TPQ, 1, H), jnp.float32),
            pltpu.SemaphoreType.DMA,
            pltpu.SemaphoreType.DMA,
        ],
    )(y, d.reshape(NW, NCC, CC), w)


def kernel(x, scores, expert_weights, top_experts, w1, w2):
    del scores
    te = top_experts.reshape(-1).astype(jnp.int32)
    ew = expert_weights.reshape(-1)
    xs, d, w = _route_dispatch(te, ew, x)
    y = _grouped_mlp(xs, w1, w2)
    return _sc_combine(y, d, w)


# speedup vs baseline: 1.3155x; 1.1341x over previous
import functools

import jax
import jax.numpy as jnp
from jax import lax
from jax.experimental import pallas as pl
from jax.experimental.pallas import tpu as pltpu
from jax.experimental.pallas import tpu_sc as plsc

E = 8
TOPK = 2
H = 1024
F = 4096
T = 4096
P = T * TOPK
CAP = 1024

NC, NS, L = 2, 16, 16
NW = NC * NS
PPW = P // NW
BLK = 2 * PPW
CH = 32
NCH = PPW // CH
TPW = T // NW
TPC = CH // TOPK


@functools.cache
def _sc_mesh():
    return plsc.VectorSubcoreMesh(core_axis_name="c", subcore_axis_name="s",
                                  num_cores=NC, num_subcores=NS)


def _prefix16(x, sb):
    acc = x
    for k in (1, 2, 4, 8):
        sb[pl.ds(L, L)] = acc
        acc = acc + sb[pl.ds(L - k, L)]
    return acc


def _route_dispatch_body(te_hbm, ew_hbm, x_hbm, xs_hbm, d_hbm, w_hbm,
                         te_v, ranks_v, ew_v, row_v, tab_v, sb_v, hist_sh,
                         dd_v, tok_v, dout_v, wout_v, rows_v, rows2_v, zrow_v,
                         gsem, ssem):
    cid = lax.axis_index("c")
    sid = lax.axis_index("s")
    g = 2 * sid + cid
    lanes = lax.iota(jnp.int32, L)
    zeros = jnp.zeros((L,), jnp.int32)

    blk_base = sid * BLK
    pltpu.sync_copy(te_hbm.at[pl.ds(blk_base, BLK)], te_v)
    sb_v[pl.ds(0, L)] = zeros
    cnt = [jnp.int32(0)] * E
    for j in range(BLK // L):
        v = te_v[pl.ds(j * L, L)]
        lo = v < 4
        sh1 = jnp.minimum(v, 3) * 8
        sh2 = jnp.maximum(v - 4, 0) * 8
        pv1 = jnp.where(lo, jnp.int32(1) << sh1, zeros)
        pv2 = jnp.where(lo, zeros, jnp.int32(1) << sh2)
        pre1 = _prefix16(pv1, sb_v)
        pre2 = _prefix16(pv2, sb_v)
        fld = jnp.where(lo, pre1 >> sh1, pre2 >> sh2) & 0xFF
        carry = jnp.full((L,), cnt[7], jnp.int32)
        for e in range(E - 1):
            carry = jnp.where(v == e, jnp.full((L,), cnt[e], jnp.int32), carry)
        ranks_v[pl.ds(j * L, L)] = fld - 1 + carry
        l1, l2 = pre1[L - 1], pre2[L - 1]
        for e in range(4):
            cnt[e] = cnt[e] + ((l1 >> (8 * e)) & 0xFF)
            cnt[e + 4] = cnt[e + 4] + ((l2 >> (8 * e)) & 0xFF)
    hist = jnp.full((L,), 0, jnp.int32)
    for e in range(E):
        hist = jnp.where(lanes == e, cnt[e], hist)
    row_v[...] = hist
    pltpu.sync_copy(row_v, hist_sh.at[sid])
    plsc.subcore_barrier()
    pltpu.sync_copy(hist_sh, tab_v)

    totals = zeros
    pfx = zeros
    sidv = jnp.full((L,), sid, jnp.int32)
    ones = jnp.full((L,), 1, jnp.int32)
    for s in range(NS):
        r = tab_v[s]
        totals = totals + r
        step = jnp.minimum(jnp.maximum(sidv - jnp.full((L,), s, jnp.int32),
                                       zeros), ones)
        pfx = pfx + r * step
    off = [None] * E
    cap_off = [None] * E
    tot0 = totals[0]
    for e in range(E):
        p_e = pfx[e]
        off[e] = e * CAP + p_e
        cap_off[e] = p_e

    loc = cid * PPW
    pltpu.sync_copy(ew_hbm.at[pl.ds(blk_base + loc, PPW)], ew_v)
    for j in range(PPW // L):
        v = te_v[pl.ds(loc + j * L, L)]
        r = ranks_v[pl.ds(loc + j * L, L)]
        ewv = ew_v[pl.ds(j * L, L)]
        offs = jnp.full((L,), off[7], jnp.int32)
        pfxs = jnp.full((L,), cap_off[7], jnp.int32)
        for e in range(E - 1):
            m = v == e
            offs = jnp.where(m, jnp.full((L,), off[e], jnp.int32), offs)
            pfxs = jnp.where(m, jnp.full((L,), cap_off[e], jnp.int32), pfxs)
        d = offs + r
        kept = (pfxs + r) < CAP
        dd = jnp.where(kept, d, jnp.full((L,), P + g, jnp.int32))
        dout = jnp.where(kept, d, jnp.zeros((L,), jnp.int32))
        wout = jnp.where(kept, ewv, jnp.zeros((L,), jnp.float32))
        tok = lax.shift_right_logical(blk_base + loc + j * L + lanes, 1)
        k, half = j // 2, (j % 2) * L
        dd_v[k, pl.ds(half, L)] = dd
        tok_v[k, pl.ds(half, L)] = tok
        dout_v[k, pl.ds(half, L)] = dout
        wout_v[pl.ds(j * L, L)] = wout
    pltpu.sync_copy(dout_v, d_hbm.at[g])
    pltpu.sync_copy(wout_v, w_hbm.at[g])

    for i in range(H // L):
        zrow_v[0, pl.ds(i * L, L)] = jnp.zeros((L,), jnp.float32)

    @pl.when(jnp.logical_and(g == 0, tot0 == 0))
    def _():
        pltpu.sync_copy(zrow_v, xs_hbm.at[pl.ds(0, 1)])

    bufs = (rows_v, rows2_v)
    gd = [None] * NCH
    sd = [None] * NCH
    gd[0] = pltpu.async_copy(x_hbm.at[tok_v.at[0]], bufs[0], gsem)
    for k in range(NCH):
        gd[k].wait()
        sd[k] = pltpu.async_copy(bufs[k % 2].at[:, 0],
                                 xs_hbm.at[dd_v.at[k]], ssem)
        if k >= 1:
            sd[k - 1].wait()
        if k + 1 < NCH:
            gd[k + 1] = pltpu.async_copy(x_hbm.at[tok_v.at[k + 1]],
                                         bufs[(k + 1) % 2], gsem)
    sd[NCH - 1].wait()


def _route_dispatch(te, ew, xf):
    return pl.kernel(
        _route_dispatch_body,
        out_type=[
            jax.ShapeDtypeStruct((P + NW, H), jnp.float32),
            jax.ShapeDtypeStruct((NW, NCH, CH), jnp.int32),
            jax.ShapeDtypeStruct((NW, PPW), jnp.float32),
        ],
        mesh=_sc_mesh(),
        scratch_types=[
            pltpu.VMEM((BLK,), jnp.int32),
            pltpu.VMEM((BLK,), jnp.int32),
            pltpu.VMEM((PPW,), jnp.float32),
            pltpu.VMEM((L,), jnp.int32),
            pltpu.VMEM((NS, L), jnp.int32),
            pltpu.VMEM((2 * L,), jnp.int32),
            pltpu.VMEM_SHARED((NS, L), jnp.int32),
            pltpu.VMEM((NCH, CH), jnp.int32),
            pltpu.VMEM((NCH, CH), jnp.int32),
            pltpu.VMEM((NCH, CH), jnp.int32),
            pltpu.VMEM((PPW,), jnp.float32),
            pltpu.VMEM((CH, 1, H), jnp.float32),
            pltpu.VMEM((CH, 1, H), jnp.float32),
            pltpu.VMEM((1, H), jnp.float32),
            pltpu.SemaphoreType.DMA,
            pltpu.SemaphoreType.DMA,
        ],
    )(te, ew, xf)


FB = 1024
FN = F // FB


def _mlp_body(xs_ref, w1_ref, w2_ref, y_ref, acc_ref):
    f = pl.program_id(1)
    a = xs_ref[...].astype(jnp.bfloat16)
    h = jnp.dot(a, w1_ref[0].astype(jnp.bfloat16),
                preferred_element_type=jnp.float32)
    hb = jax.nn.gelu(h).astype(jnp.bfloat16)
    part = jnp.dot(hb, w2_ref[0].astype(jnp.bfloat16),
                   preferred_element_type=jnp.float32)

    @pl.when(f == 0)
    def _():
        acc_ref[...] = jnp.zeros_like(acc_ref)

    acc_ref[...] += part

    @pl.when(f == FN - 1)
    def _():
        y_ref[...] = acc_ref[...]


def _grouped_mlp(xs, w1, w2):
    return pl.pallas_call(
        _mlp_body,
        grid=(E, FN),
        in_specs=[
            pl.BlockSpec((CAP, H), lambda e, f: (e, 0)),
            pl.BlockSpec((1, H, FB), lambda e, f: (e, 0, f)),
            pl.BlockSpec((1, FB, H), lambda e, f: (e, f, 0)),
        ],
        out_specs=pl.BlockSpec((CAP, H), lambda e, f: (e, 0)),
        out_shape=jax.ShapeDtypeStruct((P, H), jnp.float32),
        scratch_shapes=[pltpu.VMEM((CAP, H), jnp.float32)],
        compiler_params=pltpu.CompilerParams(
            dimension_semantics=("parallel", "arbitrary"),
        ),
    )(xs, w1, w2)


VPR = H // L


CC = 16
NCC = PPW // CC
TPQ = CC // TOPK


def _sc_combine_body(y_hbm, d_hbm, w_hbm, out_hbm, d_v, w_v, bufa_v, bufb_v,
                     out_v, sema, semb):
    cid = lax.axis_index("c")
    sid = lax.axis_index("s")
    g = 2 * sid + cid
    pltpu.sync_copy(d_hbm.at[g], d_v)
    pltpu.sync_copy(w_hbm.at[g], w_v.at[pl.ds(0, PPW)])

    def compute(buf, k):
        for t in range(TPQ):
            wv = w_v[pl.ds(k * CC + 2 * t, L)]
            w0 = wv[0]
            w1 = wv[1]
            for i in range(VPR):
                r0 = buf[2 * t, pl.ds(i * L, L)]
                r1 = buf[2 * t + 1, pl.ds(i * L, L)]
                out_v[t, 0, pl.ds(i * L, L)] = w0 * r0 + w1 * r1
        pltpu.sync_copy(out_v, out_hbm.at[pl.ds(g * TPW + k * TPQ, TPQ)])

    pltpu.async_copy(y_hbm.at[d_v.at[0]], bufa_v, sema)

    def mstep(m, carry):
        k0 = 2 * m
        k1 = 2 * m + 1
        pltpu.async_copy(y_hbm.at[d_v.at[k1]], bufb_v, semb)
        pltpu.make_async_copy(y_hbm.at[d_v.at[k0]], bufa_v, sema).wait()
        compute(bufa_v, k0)

        @pl.when(k0 + 2 < NCC)
        def _():
            pltpu.async_copy(y_hbm.at[d_v.at[k0 + 2]], bufa_v, sema)

        pltpu.make_async_copy(y_hbm.at[d_v.at[k1]], bufb_v, semb).wait()
        compute(bufb_v, k1)
        return carry

    lax.fori_loop(0, NCC // 2, mstep, jnp.int32(0))


def _sc_combine(y, d, w):
    return pl.kernel(
        _sc_combine_body,
        out_type=jax.ShapeDtypeStruct((T, 1, H), jnp.float32),
        mesh=_sc_mesh(),
        scratch_types=[
            pltpu.VMEM((NCC, CC), jnp.int32),
            pltpu.VMEM((PPW + L,), jnp.float32),
            pltpu.VMEM((CC, H), jnp.float32),
            pltpu.VMEM((CC, H), jnp.float32),
            pltpu.VMEM((TPQ, 1, H), jnp.float32),
            pltpu.SemaphoreType.DMA,
            pltpu.SemaphoreType.DMA,
        ],
    )(y, d.reshape(NW, NCC, CC), w)


def kernel(x, scores, expert_weights, top_experts, w1, w2):
    del scores
    te = top_experts.reshape(-1).astype(jnp.int32)
    ew = expert_weights.reshape(-1)
    xs, d, w = _route_dispatch(te, ew, x)
    y = _grouped_mlp(xs, w1, w2)
    return _sc_combine(y, d, w)


# speedup vs baseline: 1.3622x; 1.0355x over previous
import functools

import jax
import jax.numpy as jnp
from jax import lax
from jax.experimental import pallas as pl
from jax.experimental.pallas import tpu as pltpu
from jax.experimental.pallas import tpu_sc as plsc

E = 8
TOPK = 2
H = 1024
F = 4096
T = 4096
P = T * TOPK
CAP = 1024

NC, NS, L = 2, 16, 16
NW = NC * NS
PPW = P // NW
BLK = 2 * PPW
CH = 32
NCH = PPW // CH
TPW = T // NW
TPC = CH // TOPK


@functools.cache
def _sc_mesh():
    return plsc.VectorSubcoreMesh(core_axis_name="c", subcore_axis_name="s",
                                  num_cores=NC, num_subcores=NS)


def _prefix16(x, sb):
    acc = x
    for k in (1, 2, 4, 8):
        sb[pl.ds(L, L)] = acc
        acc = acc + sb[pl.ds(L - k, L)]
    return acc


def _route_dispatch_body(te_hbm, ew_hbm, x_hbm, xs_hbm, d_hbm, w_hbm,
                         te_v, ranks_v, ew_v, row_v, tab_v, sb_v, hist_sh,
                         dd_v, tok_v, dout_v, wout_v, rows_v, rows2_v, zrow_v,
                         gsem, ssem):
    cid = lax.axis_index("c")
    sid = lax.axis_index("s")
    g = 2 * sid + cid
    lanes = lax.iota(jnp.int32, L)
    zeros = jnp.zeros((L,), jnp.int32)

    blk_base = sid * BLK
    pltpu.sync_copy(te_hbm.at[pl.ds(blk_base, BLK)], te_v)
    sb_v[pl.ds(0, L)] = zeros
    cnt = [jnp.int32(0)] * E
    for j in range(BLK // L):
        v = te_v[pl.ds(j * L, L)]
        lo = v < 4
        sh1 = jnp.minimum(v, 3) * 8
        sh2 = jnp.maximum(v - 4, 0) * 8
        pv1 = jnp.where(lo, jnp.int32(1) << sh1, zeros)
        pv2 = jnp.where(lo, zeros, jnp.int32(1) << sh2)
        pre1 = _prefix16(pv1, sb_v)
        pre2 = _prefix16(pv2, sb_v)
        fld = jnp.where(lo, pre1 >> sh1, pre2 >> sh2) & 0xFF
        carry = jnp.full((L,), cnt[7], jnp.int32)
        for e in range(E - 1):
            carry = jnp.where(v == e, jnp.full((L,), cnt[e], jnp.int32), carry)
        ranks_v[pl.ds(j * L, L)] = fld - 1 + carry
        l1, l2 = pre1[L - 1], pre2[L - 1]
        for e in range(4):
            cnt[e] = cnt[e] + ((l1 >> (8 * e)) & 0xFF)
            cnt[e + 4] = cnt[e + 4] + ((l2 >> (8 * e)) & 0xFF)
    hist = jnp.full((L,), 0, jnp.int32)
    for e in range(E):
        hist = jnp.where(lanes == e, cnt[e], hist)
    row_v[...] = hist
    pltpu.sync_copy(row_v, hist_sh.at[sid])
    plsc.subcore_barrier()
    pltpu.sync_copy(hist_sh, tab_v)

    totals = zeros
    pfx = zeros
    sidv = jnp.full((L,), sid, jnp.int32)
    ones = jnp.full((L,), 1, jnp.int32)
    for s in range(NS):
        r = tab_v[s]
        totals = totals + r
        step = jnp.minimum(jnp.maximum(sidv - jnp.full((L,), s, jnp.int32),
                                       zeros), ones)
        pfx = pfx + r * step
    off = [None] * E
    cap_off = [None] * E
    tot0 = totals[0]
    for e in range(E):
        p_e = pfx[e]
        off[e] = e * CAP + p_e
        cap_off[e] = p_e

    loc = cid * PPW
    pltpu.sync_copy(ew_hbm.at[pl.ds(blk_base + loc, PPW)], ew_v)
    for j in range(PPW // L):
        v = te_v[pl.ds(loc + j * L, L)]
        r = ranks_v[pl.ds(loc + j * L, L)]
        ewv = ew_v[pl.ds(j * L, L)]
        offs = jnp.full((L,), off[7], jnp.int32)
        pfxs = jnp.full((L,), cap_off[7], jnp.int32)
        for e in range(E - 1):
            m = v == e
            offs = jnp.where(m, jnp.full((L,), off[e], jnp.int32), offs)
            pfxs = jnp.where(m, jnp.full((L,), cap_off[e], jnp.int32), pfxs)
        d = offs + r
        kept = (pfxs + r) < CAP
        dd = jnp.where(kept, d, jnp.full((L,), P + g, jnp.int32))
        dout = jnp.where(kept, d, jnp.zeros((L,), jnp.int32))
        wout = jnp.where(kept, ewv, jnp.zeros((L,), jnp.float32))
        tok = lax.shift_right_logical(blk_base + loc + j * L + lanes, 1)
        k, half = j // 2, (j % 2) * L
        dd_v[k, pl.ds(half, L)] = dd
        tok_v[k, pl.ds(half, L)] = tok
        dout_v[k, pl.ds(half, L)] = dout
        wout_v[pl.ds(j * L, L)] = wout
    pltpu.sync_copy(dout_v, d_hbm.at[g])
    pltpu.sync_copy(wout_v, w_hbm.at[g])

    for i in range(H // L):
        zrow_v[0, pl.ds(i * L, L)] = jnp.zeros((L,), jnp.float32)

    @pl.when(jnp.logical_and(g == 0, tot0 == 0))
    def _():
        pltpu.sync_copy(zrow_v, xs_hbm.at[pl.ds(0, 1)])

    bufs = (rows_v, rows2_v)
    gd = [None] * NCH
    sd = [None] * NCH
    gd[0] = pltpu.async_copy(x_hbm.at[tok_v.at[0]], bufs[0], gsem)
    for k in range(NCH):
        gd[k].wait()
        sd[k] = pltpu.async_copy(bufs[k % 2].at[:, 0],
                                 xs_hbm.at[dd_v.at[k]], ssem)
        if k >= 1:
            sd[k - 1].wait()
        if k + 1 < NCH:
            gd[k + 1] = pltpu.async_copy(x_hbm.at[tok_v.at[k + 1]],
                                         bufs[(k + 1) % 2], gsem)
    sd[NCH - 1].wait()


def _route_dispatch(te, ew, xf):
    return pl.kernel(
        _route_dispatch_body,
        out_type=[
            jax.ShapeDtypeStruct((P + NW, H), jnp.float32),
            jax.ShapeDtypeStruct((NW, NCH, CH), jnp.int32),
            jax.ShapeDtypeStruct((NW, PPW), jnp.float32),
        ],
        mesh=_sc_mesh(),
        scratch_types=[
            pltpu.VMEM((BLK,), jnp.int32),
            pltpu.VMEM((BLK,), jnp.int32),
            pltpu.VMEM((PPW,), jnp.float32),
            pltpu.VMEM((L,), jnp.int32),
            pltpu.VMEM((NS, L), jnp.int32),
            pltpu.VMEM((2 * L,), jnp.int32),
            pltpu.VMEM_SHARED((NS, L), jnp.int32),
            pltpu.VMEM((NCH, CH), jnp.int32),
            pltpu.VMEM((NCH, CH), jnp.int32),
            pltpu.VMEM((NCH, CH), jnp.int32),
            pltpu.VMEM((PPW,), jnp.float32),
            pltpu.VMEM((CH, 1, H), jnp.float32),
            pltpu.VMEM((CH, 1, H), jnp.float32),
            pltpu.VMEM((1, H), jnp.float32),
            pltpu.SemaphoreType.DMA,
            pltpu.SemaphoreType.DMA,
        ],
    )(te, ew, xf)


FB = 2048
FN = F // FB


def _mlp_body(xs_ref, w1_ref, w2_ref, y_ref, acc_ref):
    f = pl.program_id(1)
    a = xs_ref[...].astype(jnp.bfloat16)
    h = jnp.dot(a, w1_ref[0].astype(jnp.bfloat16),
                preferred_element_type=jnp.float32)
    hb = jax.nn.gelu(h).astype(jnp.bfloat16)
    part = jnp.dot(hb, w2_ref[0].astype(jnp.bfloat16),
                   preferred_element_type=jnp.float32)

    @pl.when(f == 0)
    def _():
        acc_ref[...] = jnp.zeros_like(acc_ref)

    acc_ref[...] += part

    @pl.when(f == FN - 1)
    def _():
        y_ref[...] = acc_ref[...]


def _grouped_mlp(xs, w1, w2):
    return pl.pallas_call(
        _mlp_body,
        grid=(E, FN),
        in_specs=[
            pl.BlockSpec((CAP, H), lambda e, f: (e, 0)),
            pl.BlockSpec((1, H, FB), lambda e, f: (e, 0, f)),
            pl.BlockSpec((1, FB, H), lambda e, f: (e, f, 0)),
        ],
        out_specs=pl.BlockSpec((CAP, H), lambda e, f: (e, 0)),
        out_shape=jax.ShapeDtypeStruct((P, H), jnp.float32),
        scratch_shapes=[pltpu.VMEM((CAP, H), jnp.float32)],
        compiler_params=pltpu.CompilerParams(
            dimension_semantics=("parallel", "arbitrary"),
        ),
    )(xs, w1, w2)


VPR = H // L


CC = 16
NCC = PPW // CC
TPQ = CC // TOPK


def _sc_combine_body(y_hbm, d_hbm, w_hbm, out_hbm, d_v, w_v, bufa_v, bufb_v,
                     out_v, sema, semb):
    cid = lax.axis_index("c")
    sid = lax.axis_index("s")
    g = 2 * sid + cid
    pltpu.sync_copy(d_hbm.at[g], d_v)
    pltpu.sync_copy(w_hbm.at[g], w_v.at[pl.ds(0, PPW)])

    def compute(buf, k):
        for t in range(TPQ):
            wv = w_v[pl.ds(k * CC + 2 * t, L)]
            w0 = wv[0]
            w1 = wv[1]
            for i in range(VPR):
                r0 = buf[2 * t, pl.ds(i * L, L)]
                r1 = buf[2 * t + 1, pl.ds(i * L, L)]
                out_v[t, 0, pl.ds(i * L, L)] = w0 * r0 + w1 * r1
        pltpu.sync_copy(out_v, out_hbm.at[pl.ds(g * TPW + k * TPQ, TPQ)])

    pltpu.async_copy(y_hbm.at[d_v.at[0]], bufa_v, sema)

    def mstep(m, carry):
        k0 = 2 * m
        k1 = 2 * m + 1
        pltpu.async_copy(y_hbm.at[d_v.at[k1]], bufb_v, semb)
        pltpu.make_async_copy(y_hbm.at[d_v.at[k0]], bufa_v, sema).wait()
        compute(bufa_v, k0)

        @pl.when(k0 + 2 < NCC)
        def _():
            pltpu.async_copy(y_hbm.at[d_v.at[k0 + 2]], bufa_v, sema)

        pltpu.make_async_copy(y_hbm.at[d_v.at[k1]], bufb_v, semb).wait()
        compute(bufb_v, k1)
        return carry

    lax.fori_loop(0, NCC // 2, mstep, jnp.int32(0))


def _sc_combine(y, d, w):
    return pl.kernel(
        _sc_combine_body,
        out_type=jax.ShapeDtypeStruct((T, 1, H), jnp.float32),
        mesh=_sc_mesh(),
        scratch_types=[
            pltpu.VMEM((NCC, CC), jnp.int32),
            pltpu.VMEM((PPW + L,), jnp.float32),
            pltpu.VMEM((CC, H), jnp.float32),
            pltpu.VMEM((CC, H), jnp.float32),
            pltpu.VMEM((TPQ, 1, H), jnp.float32),
            pltpu.SemaphoreType.DMA,
            pltpu.SemaphoreType.DMA,
        ],
    )(y, d.reshape(NW, NCC, CC), w)


def kernel(x, scores, expert_weights, top_experts, w1, w2):
    del scores
    te = top_experts.reshape(-1).astype(jnp.int32)
    ew = expert_weights.reshape(-1)
    xs, d, w = _route_dispatch(te, ew, x)
    y = _grouped_mlp(xs, w1, w2)
    return _sc_combine(y, d, w)
